# baseline (speedup 1.0000x reference)
import functools

import jax
import jax.numpy as jnp
from jax import lax
from jax.experimental import pallas as pl
from jax.experimental.pallas import tpu as pltpu
from jax.experimental.pallas import tpu_sc as plsc

N_NODES = 10000
N_EDGES = 320000
N_GRAPHS = 128

NP = 10240
BLK = 1280
GRID = NP // BLK
EPAD = 327680
NSLAB = 16
NCHUNK = 160
CHUNK = 128
ROWS_PER_TILE = NP // 16

_LN10 = 2.302585092994046


def _make_sc_agg(F2, CK=CHUNK):
    mesh = plsc.VectorSubcoreMesh(core_axis_name="c", subcore_axis_name="s",
                                  num_cores=2, num_subcores=16)
    NCH = (EPAD // NSLAB) // CK

    @functools.partial(
        pl.kernel,
        out_type=jax.ShapeDtypeStruct((2 * NP, F2), jnp.float32),
        mesh=mesh,
        compiler_params=pltpu.CompilerParams(use_tc_tiling_on_sc=False),
        scratch_types=[
            pltpu.VMEM((NCH + 2, CK), jnp.int32),
            pltpu.VMEM((NCH, CK), jnp.int32),
            pltpu.VMEM((CK, F2), jnp.float32),
            pltpu.VMEM((CK, F2), jnp.float32),
            pltpu.VMEM((16, F2), jnp.float32),
            pltpu.VMEM_SHARED((NP, F2), jnp.float32),
            pltpu.SemaphoreType.DMA,
            pltpu.SemaphoreType.DMA,
        ],
    )
    def agg(h_hbm, src_hbm, dst_hbm, out_hbm, src_v, dst_v, rows0_v, rows1_v,
            zbuf_v, acc, sem0, sem1):
        c = lax.axis_index("c")
        s = lax.axis_index("s")

        pltpu.sync_copy(src_hbm.at[c, s], src_v.at[pl.ds(0, NCH)])
        pltpu.sync_copy(dst_hbm.at[s], dst_v)
        for i in range(2):
            for j in range(CK // 16):
                src_v[NCH + i, pl.ds(j * 16, 16)] = jnp.zeros((16,), jnp.int32)

        for i in range(16):
            for j in range(F2 // 16):
                zbuf_v[i, pl.ds(j * 16, 16)] = jnp.zeros((16,), jnp.float32)

        def zloop(k, carry):
            pltpu.sync_copy(zbuf_v, acc.at[pl.ds(s * ROWS_PER_TILE + k * 16, 16)])
            return carry
        lax.fori_loop(0, ROWS_PER_TILE // 16, zloop, 0)
        plsc.subcore_barrier()

        pltpu.async_copy(h_hbm.at[src_v.at[0]], rows0_v, sem0)

        def eloop(i, carry):
            j = 2 * i
            pltpu.async_copy(h_hbm.at[src_v.at[j + 1]], rows1_v, sem1)
            pltpu.make_async_copy(h_hbm.at[src_v.at[j]], rows0_v, sem0).wait()
            pltpu.sync_copy(rows0_v, acc.at[dst_v.at[j]], add=True)
            pltpu.async_copy(h_hbm.at[src_v.at[j + 2]], rows0_v, sem0)
            pltpu.make_async_copy(h_hbm.at[src_v.at[j + 1]], rows1_v, sem1).wait()
            pltpu.sync_copy(rows1_v, acc.at[dst_v.at[j + 1]], add=True)
            return carry
        lax.fori_loop(0, NCH // 2, eloop, 0)
        pltpu.make_async_copy(h_hbm.at[src_v.at[NCH]], rows0_v, sem0).wait()
        plsc.subcore_barrier()

        def cloop(k, carry):
            r0 = s * ROWS_PER_TILE + k * 128
            pltpu.sync_copy(acc.at[pl.ds(r0, 128)], rows0_v.at[pl.ds(0, 128)])
            pltpu.sync_copy(rows0_v.at[pl.ds(0, 128)],
                            out_hbm.at[pl.ds(c * NP + r0, 128)])
            return carry
        lax.fori_loop(0, ROWS_PER_TILE // 128, cloop, 0)

    return agg


def _make_sc_deg():
    mesh = plsc.VectorSubcoreMesh(core_axis_name="c", subcore_axis_name="s",
                                  num_cores=2, num_subcores=16)

    @functools.partial(
        pl.kernel,
        out_type=jax.ShapeDtypeStruct((2 * NP, 16), jnp.float32),
        mesh=mesh,
        compiler_params=pltpu.CompilerParams(use_tc_tiling_on_sc=False),
        scratch_types=[
            pltpu.VMEM((NCHUNK, CHUNK), jnp.int32),
            pltpu.VMEM((CHUNK, 16), jnp.float32),
            pltpu.VMEM((16, 16), jnp.float32),
            pltpu.VMEM_SHARED((NP, 16), jnp.float32),
        ],
    )
    def deg(dst_hbm, out_hbm, dst_v, ones_v, zbuf_v, acc):
        c = lax.axis_index("c")
        s = lax.axis_index("s")

        pltpu.sync_copy(dst_hbm.at[s], dst_v)
        for i in range(CHUNK):
            ones_v[i, pl.ds(0, 16)] = jnp.ones((16,), jnp.float32)
        for i in range(16):
            zbuf_v[i, pl.ds(0, 16)] = jnp.zeros((16,), jnp.float32)

        def zloop(k, carry):
            pltpu.sync_copy(zbuf_v, acc.at[pl.ds(s * ROWS_PER_TILE + k * 16, 16)])
            return carry
        lax.fori_loop(0, ROWS_PER_TILE // 16, zloop, 0)
        plsc.subcore_barrier()

        def eloop(j, carry):
            pltpu.sync_copy(ones_v, acc.at[dst_v.at[j]], add=True)
            return carry
        lax.fori_loop(c * (NCHUNK // 2), (c + 1) * (NCHUNK // 2), eloop, 0)
        plsc.subcore_barrier()

        def cloop(k, carry):
            r0 = s * ROWS_PER_TILE + k * CHUNK
            pltpu.sync_copy(acc.at[pl.ds(r0, CHUNK)], ones_v)
            pltpu.sync_copy(ones_v, out_hbm.at[pl.ds(c * NP + r0, CHUNK)])
            return carry
        lax.fori_loop(0, ROWS_PER_TILE // CHUNK, cloop, 0)

    return deg


_SC_AGG_CACHE = {}


def _sc_agg(F2, CK=CHUNK):
    if (F2, CK) not in _SC_AGG_CACHE:
        _SC_AGG_CACHE[(F2, CK)] = _make_sc_agg(F2, CK)
    return _SC_AGG_CACHE[(F2, CK)]


def _sc_deg():
    if 'deg' not in _SC_AGG_CACHE:
        _SC_AGG_CACHE['deg'] = _make_sc_deg()
    return _SC_AGG_CACHE['deg']


def _stack_halves(h, lo1, F2):
    return jnp.concatenate([h[:, :F2], h[:, lo1:lo1 + F2]], axis=0)


def _row_spec(F):
    return pl.BlockSpec((BLK, F), lambda i: (i, 0))


def _p_specs(F2):
    return (pl.BlockSpec((BLK, F2), lambda i: (i, 0)),
            pl.BlockSpec((BLK, F2), lambda i: (i + GRID, 0)))


def _full_spec(shape):
    nd = len(shape)
    return pl.BlockSpec(shape, lambda *_: (0,) * nd)


def _relu(v):
    return jnp.maximum(v, 0.0)


def _recomb(p0, p1, keep0):
    return jnp.concatenate([p0[:, :keep0], p1], axis=1)


def _tca_body(p0, p1, x, dinv_o, xs_o):
    deg = p0[:, 0:1] + p1[:, 0:1] + 1.0
    dinv = lax.rsqrt(deg)
    dinv_o[...] = dinv
    xs_o[...] = x[...] * dinv


def _tca(pdeg, x):
    s0, s1 = _p_specs(16)
    return pl.pallas_call(
        _tca_body,
        grid=(GRID,),
        in_specs=[s0, s1, _row_spec(128)],
        out_specs=[_row_spec(1), _row_spec(128)],
        out_shape=[jax.ShapeDtypeStruct((NP, 1), jnp.float32),
                   jax.ShapeDtypeStruct((NP, 128), jnp.float32)],
    )(pdeg, pdeg, x)


def _tcb_body(p0, p1, xs, dinv, wc, bc, hsat_o, h1s_o):
    dv = dinv[...]
    aggx = dv * (_recomb(p0[...], p1[...], 64) + xs[...])
    hcat = _relu(jnp.dot(aggx, wc[...], preferred_element_type=jnp.float32)
                 + bc[...])
    hsat_o[...] = hcat[:, :64]
    h1s_o[...] = dv * hcat[:, 64:]


def _tcb(pa, xs, dinv, wc, bc):
    s0, s1 = _p_specs(64)
    return pl.pallas_call(
        _tcb_body,
        grid=(GRID,),
        in_specs=[s0, s1, _row_spec(128), _row_spec(1),
                  _full_spec((128, 176)), _full_spec((1, 176))],
        out_specs=[_row_spec(64), _row_spec(112)],
        out_shape=[jax.ShapeDtypeStruct((NP, 64), jnp.float32),
                   jax.ShapeDtypeStruct((NP, 112), jnp.float32)],
    )(pa, pa, xs, dinv, wc, bc)


def _tcc_body(p0, p1, h1s, dinv, w1, b1, w2, t2s_o):
    dv = dinv[...]
    agg1 = dv * (_recomb(p0[...], p1[...], 48) + h1s[...])
    h2 = _relu(jnp.dot(agg1, w1[...], preferred_element_type=jnp.float32)
               + b1[...])
    t2s_o[...] = dv * jnp.dot(h2, w2[...], preferred_element_type=jnp.float32)


def _tcc(pb, h1s, dinv, w1, b1, w2):
    s0, s1 = _p_specs(64)
    return pl.pallas_call(
        _tcc_body,
        grid=(GRID,),
        in_specs=[s0, s1, _row_spec(112), _row_spec(1),
                  _full_spec((112, 420)), _full_spec((1, 420)),
                  _full_spec((420, 144))],
        out_specs=_row_spec(144),
        out_shape=jax.ShapeDtypeStruct((NP, 144), jnp.float32),
    )(pb, pb, h1s, dinv, w1, b1, w2)


def _tcd_body(p0, p1, t2s, dinv, b2, w3, t3s_o):
    dv = dinv[...]
    h3 = _relu(dv * (_recomb(p0[...], p1[...], 64) + t2s[...]) + b2[...])
    t3s_o[...] = dv * jnp.dot(h3, w3[...], preferred_element_type=jnp.float32)


def _tcd(pc, t2s, dinv, b2, w3):
    s0, s1 = _p_specs(80)
    return pl.pallas_call(
        _tcd_body,
        grid=(GRID,),
        in_specs=[s0, s1, _row_spec(144), _row_spec(1),
                  _full_spec((1, 144)), _full_spec((144, 144))],
        out_specs=_row_spec(144),
        out_shape=jax.ShapeDtypeStruct((NP, 144), jnp.float32),
    )(pc, pc, t2s, dinv, b2, w3)


def _tce_body(p0, p1, t3s, dinv, b3, hsat, bm, g_o):
    @pl.when(pl.program_id(0) == 0)
    def _():
        g_o[...] = jnp.zeros_like(g_o)

    h4 = _relu(dinv[...] * (_recomb(p0[...], p1[...], 64) + t3s[...])
               + b3[...])
    hcat = jnp.concatenate([hsat[...], h4], axis=1)
    ids = lax.broadcasted_iota(jnp.int32, (BLK, N_GRAPHS), 1).astype(jnp.float32)
    onehot = (bm[...] == ids).astype(jnp.float32)
    g_o[...] += lax.dot_general(onehot, hcat, (((0,), (0,)), ((), ())),
                                preferred_element_type=jnp.float32)


def _tce(pd, t3s, dinv, b3, hsat, bm):
    s0, s1 = _p_specs(80)
    return pl.pallas_call(
        _tce_body,
        grid=(GRID,),
        in_specs=[s0, s1, _row_spec(144), _row_spec(1),
                  _full_spec((1, 144)), _row_spec(64), _row_spec(1)],
        out_specs=_full_spec((N_GRAPHS, 208)),
        out_shape=jax.ShapeDtypeStruct((N_GRAPHS, 208), jnp.float32),
    )(pd, pd, t3s, dinv, b3, hsat, bm)


def _tcf_body(g, sat_wo, sat_bo, dw0, db0, dw1, db1, dw2, db2, dw3, db3,
              cw, cb, temp, xw, xa, pt, out_o):
    gs = _relu(g[:, :64])
    logp = (jnp.dot(gs, sat_wo[...], preferred_element_type=jnp.float32)
            + sat_bo[...]) * 0.8 + 2.5
    ln_p_amine = _LN10 * logp

    h = _relu(g[:, 64:])
    h = _relu(jnp.dot(h, dw0[...], preferred_element_type=jnp.float32) + db0[...])
    h = _relu(jnp.dot(h, dw1[...], preferred_element_type=jnp.float32) + db1[...])
    h = _relu(jnp.dot(h, dw2[...], preferred_element_type=jnp.float32) + db2[...])
    h = _relu(jnp.dot(h, dw3[...], preferred_element_type=jnp.float32) + db3[...])
    c = jnp.dot(h, cw[...], preferred_element_type=jnp.float32) + cb[...]
    c0, c1, c2 = c[:, 0:1], c[:, 1:2], c[:, 2:3]

    t = temp[...]
    x_w = xw[...]
    x_a = xa[...]

    sig = 1.0 / (1.0 + jnp.exp(-c0))
    alpha = 0.2 * (1.0 + sig / 10.0 * (0.47 / 0.2))
    rtk = 62.36367 * (t + 273.15)
    tau12 = c1 / rtk
    tau21 = c2 / rtk
    g12 = jnp.exp(-tau12 * alpha)
    g21 = jnp.exp(-tau21 * alpha)
    den21 = x_a + x_w * g21
    den12 = x_w + x_a * g12
    r21 = g21 / den21
    r12 = g12 / den12
    lga = x_w * x_w * (tau21 * r21 * r21 + tau12 * g12 / (den12 * den12))
    lgw = x_a * x_a * (tau12 * r12 * r12 + tau21 * g21 / (den21 * den21))

    ln_p_water = _LN10 * jnp.where(
        t < 100.0,
        8.07131 - 1730.63 / (t + 233.426),
        8.14019 - 1810.94 / (t + 244.485))
    ln_p_tot = jnp.log(pt[...] / 133.322)

    ln_y_amine = jnp.log(x_a) + lga + ln_p_amine - ln_p_tot
    ln_y_water = jnp.log(x_w) + lgw + ln_p_water - ln_p_tot
    out_o[...] = jnp.concatenate(
        [ln_y_amine, ln_y_water, ln_y_amine + ln_y_water], axis=1)


def _tcf(g, sat_wo, sat_bo, dws, dbs, cw, cb, temp, xw, xa, pt):
    args = [g, sat_wo, sat_bo]
    for w, b in zip(dws, dbs):
        args += [w, b]
    args += [cw, cb, temp, xw, xa, pt]
    return pl.pallas_call(
        _tcf_body,
        in_specs=[_full_spec(a.shape) for a in args],
        out_specs=_full_spec((N_GRAPHS, 3)),
        out_shape=jax.ShapeDtypeStruct((N_GRAPHS, 3), jnp.float32),
    )(*args)


def _pad2(w, rows, cols):
    return jnp.zeros((rows, cols), jnp.float32).at[:w.shape[0], :w.shape[1]].set(w)


def kernel(x, edge_index, batch_mapping, x_water, x_amine, y_water, y_amine,
           temperature, P_tot, params):
    f32 = jnp.float32

    xp = jnp.zeros((NP, 128), f32).at[:N_NODES].set(x)
    bm = jnp.full((NP, 1), 999.0, f32).at[:N_NODES, 0].set(
        batch_mapping.astype(f32))

    src = jnp.zeros((EPAD,), jnp.int32).at[:N_EDGES].set(edge_index[0])
    src = src.reshape(NSLAB, NCHUNK, CHUNK)
    src2 = jnp.stack([src, src + NP])
    dst_pad = N_NODES + (jnp.arange(EPAD - N_EDGES, dtype=jnp.int32)
                         % (NP - N_NODES))
    dst = jnp.concatenate([edge_index[1], dst_pad]).reshape(NSLAB, NCHUNK, CHUNK)

    pdeg = _sc_deg()(dst)
    srcw = src2.reshape(2, NSLAB, 80, 256)
    dstw = dst.reshape(NSLAB, 80, 256)
    dinv, xs = _tca(pdeg, xp)

    pa = _sc_agg(64, 256)(_stack_halves(xs, 64, 64), srcw, dstw)
    wc = jnp.concatenate(
        [params['sat_W'], _pad2(params['gcn_W0'], 128, 112)], axis=1)
    bc = jnp.concatenate(
        [params['sat_b'], jnp.zeros((112,), f32).at[:100].set(params['gcn_b0'])]
    ).reshape(1, 176)
    hsat, h1s = _tcb(pa, xs, dinv, wc, bc)

    pb = _sc_agg(64, 256)(_stack_halves(h1s, 48, 64), srcw, dstw)
    w1 = _pad2(params['gcn_W1'], 112, 420)
    b1 = params['gcn_b1'].reshape(1, 420)
    w2 = _pad2(params['gcn_W2'], 420, 144)
    t2s = _tcc(pb, h1s, dinv, w1, b1, w2)

    pc = _sc_agg(80)(_stack_halves(t2s, 64, 80), src2, dst)
    b2 = jnp.zeros((1, 144), f32).at[0, :140].set(params['gcn_b2'])
    w3 = _pad2(params['gcn_W3'], 144, 144)
    t3s = _tcd(pc, t2s, dinv, b2, w3)

    pd = _sc_agg(80)(_stack_halves(t3s, 64, 80), src2, dst)
    b3 = jnp.zeros((1, 144), f32).at[0, :140].set(params['gcn_b3'])
    g = _tce(pd, t3s, dinv, b3, hsat, bm)

    dws = [_pad2(params['dense_W0'], 144, 260), params['dense_W1'],
           params['dense_W2'], params['dense_W3']]
    dbs = [params['dense_b0'].reshape(1, 260), params['dense_b1'].reshape(1, 60),
           params['dense_b2'].reshape(1, 180), params['dense_b3'].reshape(1, 100)]
    col = lambda v: v.reshape(N_GRAPHS, 1)
    return _tcf(g, params['sat_Wo'], params['sat_bo'].reshape(1, 1),
                dws, dbs, params['coef_W'], params['coef_b'].reshape(1, 3),
                col(temperature), col(x_water), col(x_amine), col(P_tot))


# speedup vs baseline: 1.1663x; 1.1663x over previous
import functools

import jax
import jax.numpy as jnp
from jax import lax
from jax.experimental import pallas as pl
from jax.experimental.pallas import tpu as pltpu
from jax.experimental.pallas import tpu_sc as plsc

N_NODES = 10000
N_EDGES = 320000
N_GRAPHS = 128

NP = 10240
BLK = 1280
GRID = NP // BLK
EPAD = 327680
NSLAB = 16
NCHUNK = 320
CHUNK = 64
ROWS_PER_TILE = NP // 16

_LN10 = 2.302585092994046


def _make_sc_agg(F2, CK=CHUNK):
    mesh = plsc.VectorSubcoreMesh(core_axis_name="c", subcore_axis_name="s",
                                  num_cores=2, num_subcores=16)
    NCH = (EPAD // NSLAB) // CK

    @functools.partial(
        pl.kernel,
        out_type=jax.ShapeDtypeStruct((2 * NP, F2), jnp.float32),
        mesh=mesh,
        compiler_params=pltpu.CompilerParams(use_tc_tiling_on_sc=False),
        scratch_types=[
            pltpu.VMEM((NCH + 2, CK), jnp.int32),
            pltpu.VMEM((NCH, CK), jnp.int32),
            pltpu.VMEM((CK, F2), jnp.float32),
            pltpu.VMEM((CK, F2), jnp.float32),
            pltpu.VMEM((16, F2), jnp.float32),
            pltpu.VMEM_SHARED((NP, F2), jnp.float32),
            pltpu.SemaphoreType.DMA,
            pltpu.SemaphoreType.DMA,
        ],
    )
    def agg(h_hbm, src_hbm, dst_hbm, out_hbm, src_v, dst_v, rows0_v, rows1_v,
            zbuf_v, acc, sem0, sem1):
        c = lax.axis_index("c")
        s = lax.axis_index("s")

        pltpu.sync_copy(src_hbm.at[c, s], src_v.at[pl.ds(0, NCH)])
        pltpu.sync_copy(dst_hbm.at[s], dst_v)
        for i in range(2):
            for j in range(CK // 16):
                src_v[NCH + i, pl.ds(j * 16, 16)] = jnp.zeros((16,), jnp.int32)

        for i in range(16):
            for j in range(F2 // 16):
                zbuf_v[i, pl.ds(j * 16, 16)] = jnp.zeros((16,), jnp.float32)

        def zloop(k, carry):
            pltpu.sync_copy(zbuf_v, acc.at[pl.ds(s * ROWS_PER_TILE + k * 16, 16)])
            return carry
        lax.fori_loop(0, ROWS_PER_TILE // 16, zloop, 0)
        plsc.subcore_barrier()

        pltpu.async_copy(h_hbm.at[src_v.at[0]], rows0_v, sem0)

        def eloop(i, carry):
            j = 2 * i
            pltpu.async_copy(h_hbm.at[src_v.at[j + 1]], rows1_v, sem1)
            pltpu.make_async_copy(h_hbm.at[src_v.at[j]], rows0_v, sem0).wait()
            pltpu.sync_copy(rows0_v, acc.at[dst_v.at[j]], add=True)
            pltpu.async_copy(h_hbm.at[src_v.at[j + 2]], rows0_v, sem0)
            pltpu.make_async_copy(h_hbm.at[src_v.at[j + 1]], rows1_v, sem1).wait()
            pltpu.sync_copy(rows1_v, acc.at[dst_v.at[j + 1]], add=True)
            return carry
        lax.fori_loop(0, NCH // 2, eloop, 0)
        pltpu.make_async_copy(h_hbm.at[src_v.at[NCH]], rows0_v, sem0).wait()
        plsc.subcore_barrier()

        def cloop(k, carry):
            r0 = s * ROWS_PER_TILE + k * 128
            pltpu.sync_copy(acc.at[pl.ds(r0, 128)], rows0_v.at[pl.ds(0, 128)])
            pltpu.sync_copy(rows0_v.at[pl.ds(0, 128)],
                            out_hbm.at[pl.ds(c * NP + r0, 128)])
            return carry
        lax.fori_loop(0, ROWS_PER_TILE // 128, cloop, 0)

    return agg


def _make_sc_deg():
    mesh = plsc.VectorSubcoreMesh(core_axis_name="c", subcore_axis_name="s",
                                  num_cores=2, num_subcores=16)

    @functools.partial(
        pl.kernel,
        out_type=jax.ShapeDtypeStruct((2 * NP, 16), jnp.float32),
        mesh=mesh,
        compiler_params=pltpu.CompilerParams(use_tc_tiling_on_sc=False),
        scratch_types=[
            pltpu.VMEM((NCHUNK, CHUNK), jnp.int32),
            pltpu.VMEM((CHUNK, 16), jnp.float32),
            pltpu.VMEM((16, 16), jnp.float32),
            pltpu.VMEM_SHARED((NP, 16), jnp.float32),
        ],
    )
    def deg(dst_hbm, out_hbm, dst_v, ones_v, zbuf_v, acc):
        c = lax.axis_index("c")
        s = lax.axis_index("s")

        pltpu.sync_copy(dst_hbm.at[s], dst_v)
        for i in range(CHUNK):
            ones_v[i, pl.ds(0, 16)] = jnp.ones((16,), jnp.float32)
        for i in range(16):
            zbuf_v[i, pl.ds(0, 16)] = jnp.zeros((16,), jnp.float32)

        def zloop(k, carry):
            pltpu.sync_copy(zbuf_v, acc.at[pl.ds(s * ROWS_PER_TILE + k * 16, 16)])
            return carry
        lax.fori_loop(0, ROWS_PER_TILE // 16, zloop, 0)
        plsc.subcore_barrier()

        def eloop(j, carry):
            pltpu.sync_copy(ones_v, acc.at[dst_v.at[j]], add=True)
            return carry
        lax.fori_loop(c * (NCHUNK // 2), (c + 1) * (NCHUNK // 2), eloop, 0)
        plsc.subcore_barrier()

        def cloop(k, carry):
            r0 = s * ROWS_PER_TILE + k * CHUNK
            pltpu.sync_copy(acc.at[pl.ds(r0, CHUNK)], ones_v)
            pltpu.sync_copy(ones_v, out_hbm.at[pl.ds(c * NP + r0, CHUNK)])
            return carry
        lax.fori_loop(0, ROWS_PER_TILE // CHUNK, cloop, 0)

    return deg


_SC_AGG_CACHE = {}


def _sc_agg(F2, CK=CHUNK):
    if (F2, CK) not in _SC_AGG_CACHE:
        _SC_AGG_CACHE[(F2, CK)] = _make_sc_agg(F2, CK)
    return _SC_AGG_CACHE[(F2, CK)]


def _sc_deg():
    if 'deg' not in _SC_AGG_CACHE:
        _SC_AGG_CACHE['deg'] = _make_sc_deg()
    return _SC_AGG_CACHE['deg']


def _stack_halves(h, lo1, F2):
    return jnp.concatenate([h[:, :F2], h[:, lo1:lo1 + F2]], axis=0)


def _row_spec(F):
    return pl.BlockSpec((BLK, F), lambda i: (i, 0))


def _p_specs(F2):
    return (pl.BlockSpec((BLK, F2), lambda i: (i, 0)),
            pl.BlockSpec((BLK, F2), lambda i: (i + GRID, 0)))


def _full_spec(shape):
    nd = len(shape)
    return pl.BlockSpec(shape, lambda *_: (0,) * nd)


def _relu(v):
    return jnp.maximum(v, 0.0)


def _recomb(p0, p1, keep0):
    return jnp.concatenate([p0[:, :keep0], p1], axis=1)


def _tca_body(p0, p1, x, dinv_o, xs_o):
    deg = p0[:, 0:1] + p1[:, 0:1] + 1.0
    dinv = lax.rsqrt(deg)
    dinv_o[...] = dinv
    xs_o[...] = x[...] * dinv


def _tca(pdeg, x):
    s0, s1 = _p_specs(16)
    return pl.pallas_call(
        _tca_body,
        grid=(GRID,),
        in_specs=[s0, s1, _row_spec(128)],
        out_specs=[_row_spec(1), _row_spec(128)],
        out_shape=[jax.ShapeDtypeStruct((NP, 1), jnp.float32),
                   jax.ShapeDtypeStruct((NP, 128), jnp.float32)],
    )(pdeg, pdeg, x)


def _tcb_body(p0, p1, xs, dinv, wc, bc, hsat_o, h1s_o):
    dv = dinv[...]
    aggx = dv * (_recomb(p0[...], p1[...], 64) + xs[...])
    hcat = _relu(jnp.dot(aggx, wc[...], preferred_element_type=jnp.float32)
                 + bc[...])
    hsat_o[...] = hcat[:, :64]
    h1s_o[...] = dv * hcat[:, 64:]


def _tcb(pa, xs, dinv, wc, bc):
    s0, s1 = _p_specs(64)
    return pl.pallas_call(
        _tcb_body,
        grid=(GRID,),
        in_specs=[s0, s1, _row_spec(128), _row_spec(1),
                  _full_spec((128, 176)), _full_spec((1, 176))],
        out_specs=[_row_spec(64), _row_spec(112)],
        out_shape=[jax.ShapeDtypeStruct((NP, 64), jnp.float32),
                   jax.ShapeDtypeStruct((NP, 112), jnp.float32)],
    )(pa, pa, xs, dinv, wc, bc)


def _tcc_body(p0, p1, h1s, dinv, w1, b1, w2, t2s_o):
    dv = dinv[...]
    agg1 = dv * (_recomb(p0[...], p1[...], 48) + h1s[...])
    h2 = _relu(jnp.dot(agg1, w1[...], preferred_element_type=jnp.float32)
               + b1[...])
    t2s_o[...] = dv * jnp.dot(h2, w2[...], preferred_element_type=jnp.float32)


def _tcc(pb, h1s, dinv, w1, b1, w2):
    s0, s1 = _p_specs(64)
    return pl.pallas_call(
        _tcc_body,
        grid=(GRID,),
        in_specs=[s0, s1, _row_spec(112), _row_spec(1),
                  _full_spec((112, 420)), _full_spec((1, 420)),
                  _full_spec((420, 144))],
        out_specs=_row_spec(144),
        out_shape=jax.ShapeDtypeStruct((NP, 144), jnp.float32),
    )(pb, pb, h1s, dinv, w1, b1, w2)


def _tcd_body(p0, p1, t2s, dinv, b2, w3, t3s_o):
    dv = dinv[...]
    h3 = _relu(dv * (_recomb(p0[...], p1[...], 64) + t2s[...]) + b2[...])
    t3s_o[...] = dv * jnp.dot(h3, w3[...], preferred_element_type=jnp.float32)


def _tcd(pc, t2s, dinv, b2, w3):
    s0, s1 = _p_specs(80)
    return pl.pallas_call(
        _tcd_body,
        grid=(GRID,),
        in_specs=[s0, s1, _row_spec(144), _row_spec(1),
                  _full_spec((1, 144)), _full_spec((144, 144))],
        out_specs=_row_spec(144),
        out_shape=jax.ShapeDtypeStruct((NP, 144), jnp.float32),
    )(pc, pc, t2s, dinv, b2, w3)


def _tce_body(p0, p1, t3s, dinv, b3, hsat, bm, g_o):
    @pl.when(pl.program_id(0) == 0)
    def _():
        g_o[...] = jnp.zeros_like(g_o)

    h4 = _relu(dinv[...] * (_recomb(p0[...], p1[...], 64) + t3s[...])
               + b3[...])
    hcat = jnp.concatenate([hsat[...], h4], axis=1)
    ids = lax.broadcasted_iota(jnp.int32, (BLK, N_GRAPHS), 1).astype(jnp.float32)
    onehot = (bm[...] == ids).astype(jnp.float32)
    g_o[...] += lax.dot_general(onehot, hcat, (((0,), (0,)), ((), ())),
                                preferred_element_type=jnp.float32)


def _tce(pd, t3s, dinv, b3, hsat, bm):
    s0, s1 = _p_specs(80)
    return pl.pallas_call(
        _tce_body,
        grid=(GRID,),
        in_specs=[s0, s1, _row_spec(144), _row_spec(1),
                  _full_spec((1, 144)), _row_spec(64), _row_spec(1)],
        out_specs=_full_spec((N_GRAPHS, 208)),
        out_shape=jax.ShapeDtypeStruct((N_GRAPHS, 208), jnp.float32),
    )(pd, pd, t3s, dinv, b3, hsat, bm)


def _tcf_body(g, sat_wo, sat_bo, dw0, db0, dw1, db1, dw2, db2, dw3, db3,
              cw, cb, temp, xw, xa, pt, out_o):
    gs = _relu(g[:, :64])
    logp = (jnp.dot(gs, sat_wo[...], preferred_element_type=jnp.float32)
            + sat_bo[...]) * 0.8 + 2.5
    ln_p_amine = _LN10 * logp

    h = _relu(g[:, 64:])
    h = _relu(jnp.dot(h, dw0[...], preferred_element_type=jnp.float32) + db0[...])
    h = _relu(jnp.dot(h, dw1[...], preferred_element_type=jnp.float32) + db1[...])
    h = _relu(jnp.dot(h, dw2[...], preferred_element_type=jnp.float32) + db2[...])
    h = _relu(jnp.dot(h, dw3[...], preferred_element_type=jnp.float32) + db3[...])
    c = jnp.dot(h, cw[...], preferred_element_type=jnp.float32) + cb[...]
    c0, c1, c2 = c[:, 0:1], c[:, 1:2], c[:, 2:3]

    t = temp[...]
    x_w = xw[...]
    x_a = xa[...]

    sig = 1.0 / (1.0 + jnp.exp(-c0))
    alpha = 0.2 * (1.0 + sig / 10.0 * (0.47 / 0.2))
    rtk = 62.36367 * (t + 273.15)
    tau12 = c1 / rtk
    tau21 = c2 / rtk
    g12 = jnp.exp(-tau12 * alpha)
    g21 = jnp.exp(-tau21 * alpha)
    den21 = x_a + x_w * g21
    den12 = x_w + x_a * g12
    r21 = g21 / den21
    r12 = g12 / den12
    lga = x_w * x_w * (tau21 * r21 * r21 + tau12 * g12 / (den12 * den12))
    lgw = x_a * x_a * (tau12 * r12 * r12 + tau21 * g21 / (den21 * den21))

    ln_p_water = _LN10 * jnp.where(
        t < 100.0,
        8.07131 - 1730.63 / (t + 233.426),
        8.14019 - 1810.94 / (t + 244.485))
    ln_p_tot = jnp.log(pt[...] / 133.322)

    ln_y_amine = jnp.log(x_a) + lga + ln_p_amine - ln_p_tot
    ln_y_water = jnp.log(x_w) + lgw + ln_p_water - ln_p_tot
    out_o[...] = jnp.concatenate(
        [ln_y_amine, ln_y_water, ln_y_amine + ln_y_water], axis=1)


def _tcf(g, sat_wo, sat_bo, dws, dbs, cw, cb, temp, xw, xa, pt):
    args = [g, sat_wo, sat_bo]
    for w, b in zip(dws, dbs):
        args += [w, b]
    args += [cw, cb, temp, xw, xa, pt]
    return pl.pallas_call(
        _tcf_body,
        in_specs=[_full_spec(a.shape) for a in args],
        out_specs=_full_spec((N_GRAPHS, 3)),
        out_shape=jax.ShapeDtypeStruct((N_GRAPHS, 3), jnp.float32),
    )(*args)


def _pad2(w, rows, cols):
    return jnp.zeros((rows, cols), jnp.float32).at[:w.shape[0], :w.shape[1]].set(w)


def kernel(x, edge_index, batch_mapping, x_water, x_amine, y_water, y_amine,
           temperature, P_tot, params):
    f32 = jnp.float32

    xp = jnp.zeros((NP, 128), f32).at[:N_NODES].set(x)
    bm = jnp.full((NP, 1), 999.0, f32).at[:N_NODES, 0].set(
        batch_mapping.astype(f32))

    src = jnp.zeros((EPAD,), jnp.int32).at[:N_EDGES].set(edge_index[0])
    src = src.reshape(NSLAB, NCHUNK, CHUNK)
    src2 = jnp.stack([src, src + NP])
    dst_pad = N_NODES + (jnp.arange(EPAD - N_EDGES, dtype=jnp.int32)
                         % (NP - N_NODES))
    dst = jnp.concatenate([edge_index[1], dst_pad]).reshape(NSLAB, NCHUNK, CHUNK)

    pdeg = _sc_deg()(dst)
    dinv, xs = _tca(pdeg, xp)

    pa = _sc_agg(64)(_stack_halves(xs, 64, 64), src2, dst)
    wc = jnp.concatenate(
        [params['sat_W'], _pad2(params['gcn_W0'], 128, 112)], axis=1)
    bc = jnp.concatenate(
        [params['sat_b'], jnp.zeros((112,), f32).at[:100].set(params['gcn_b0'])]
    ).reshape(1, 176)
    hsat, h1s = _tcb(pa, xs, dinv, wc, bc)

    pb = _sc_agg(64)(_stack_halves(h1s, 48, 64), src2, dst)
    w1 = _pad2(params['gcn_W1'], 112, 420)
    b1 = params['gcn_b1'].reshape(1, 420)
    w2 = _pad2(params['gcn_W2'], 420, 144)
    t2s = _tcc(pb, h1s, dinv, w1, b1, w2)

    pc = _sc_agg(80)(_stack_halves(t2s, 64, 80), src2, dst)
    b2 = jnp.zeros((1, 144), f32).at[0, :140].set(params['gcn_b2'])
    w3 = _pad2(params['gcn_W3'], 144, 144)
    t3s = _tcd(pc, t2s, dinv, b2, w3)

    pd = _sc_agg(80)(_stack_halves(t3s, 64, 80), src2, dst)
    b3 = jnp.zeros((1, 144), f32).at[0, :140].set(params['gcn_b3'])
    g = _tce(pd, t3s, dinv, b3, hsat, bm)

    dws = [_pad2(params['dense_W0'], 144, 260), params['dense_W1'],
           params['dense_W2'], params['dense_W3']]
    dbs = [params['dense_b0'].reshape(1, 260), params['dense_b1'].reshape(1, 60),
           params['dense_b2'].reshape(1, 180), params['dense_b3'].reshape(1, 100)]
    col = lambda v: v.reshape(N_GRAPHS, 1)
    return _tcf(g, params['sat_Wo'], params['sat_bo'].reshape(1, 1),
                dws, dbs, params['coef_W'], params['coef_b'].reshape(1, 3),
                col(temperature), col(x_water), col(x_amine), col(P_tot))


# speedup vs baseline: 1.1816x; 1.0132x over previous
import functools

import jax
import jax.numpy as jnp
from jax import lax
from jax.experimental import pallas as pl
from jax.experimental.pallas import tpu as pltpu
from jax.experimental.pallas import tpu_sc as plsc

N_NODES = 10000
N_EDGES = 320000
N_GRAPHS = 128

NP = 10240
BLK = 1280
GRID = NP // BLK
EPAD = 327680
NSLAB = 16
NCHUNK = 160
CHUNK = 128
ROWS_PER_TILE = NP // 16

_LN10 = 2.302585092994046


def _make_sc_agg(F2):
    mesh = plsc.VectorSubcoreMesh(core_axis_name="c", subcore_axis_name="s",
                                  num_cores=2, num_subcores=16)

    @functools.partial(
        pl.kernel,
        out_type=jax.ShapeDtypeStruct((2 * NP, F2), jnp.float32),
        mesh=mesh,
        compiler_params=pltpu.CompilerParams(use_tc_tiling_on_sc=False),
        scratch_types=[
            pltpu.VMEM((NCHUNK + 2, CHUNK), jnp.int32),
            pltpu.VMEM((NCHUNK, CHUNK), jnp.int32),
            pltpu.VMEM((CHUNK, F2), jnp.float32),
            pltpu.VMEM((CHUNK, F2), jnp.float32),
            pltpu.VMEM((16, F2), jnp.float32),
            pltpu.VMEM_SHARED((NP, F2), jnp.float32),
            pltpu.SemaphoreType.DMA,
            pltpu.SemaphoreType.DMA,
        ],
    )
    def agg(h_hbm, src_hbm, dst_hbm, out_hbm, src_v, dst_v, rows0_v, rows1_v,
            zbuf_v, acc, sem0, sem1):
        c = lax.axis_index("c")
        s = lax.axis_index("s")

        pltpu.sync_copy(src_hbm.at[c, s], src_v.at[pl.ds(0, NCHUNK)])
        pltpu.sync_copy(dst_hbm.at[s], dst_v)
        for i in range(2):
            for j in range(CHUNK // 16):
                src_v[NCHUNK + i, pl.ds(j * 16, 16)] = jnp.zeros((16,), jnp.int32)

        for i in range(16):
            for j in range(F2 // 16):
                zbuf_v[i, pl.ds(j * 16, 16)] = jnp.zeros((16,), jnp.float32)

        def zloop(k, carry):
            pltpu.sync_copy(zbuf_v, acc.at[pl.ds(s * ROWS_PER_TILE + k * 16, 16)])
            return carry
        lax.fori_loop(0, ROWS_PER_TILE // 16, zloop, 0)
        plsc.subcore_barrier()

        pltpu.async_copy(h_hbm.at[src_v.at[0]], rows0_v, sem0)

        def eloop(i, carry):
            j = 2 * i
            pltpu.async_copy(h_hbm.at[src_v.at[j + 1]], rows1_v, sem1)
            pltpu.make_async_copy(h_hbm.at[src_v.at[j]], rows0_v, sem0).wait()
            pltpu.sync_copy(rows0_v, acc.at[dst_v.at[j]], add=True)
            pltpu.async_copy(h_hbm.at[src_v.at[j + 2]], rows0_v, sem0)
            pltpu.make_async_copy(h_hbm.at[src_v.at[j + 1]], rows1_v, sem1).wait()
            pltpu.sync_copy(rows1_v, acc.at[dst_v.at[j + 1]], add=True)
            return carry
        lax.fori_loop(0, NCHUNK // 2, eloop, 0)
        pltpu.make_async_copy(h_hbm.at[src_v.at[NCHUNK]], rows0_v, sem0).wait()
        plsc.subcore_barrier()

        def cloop(k, carry):
            r0 = s * ROWS_PER_TILE + k * CHUNK
            pltpu.sync_copy(acc.at[pl.ds(r0, CHUNK)], rows0_v)
            pltpu.sync_copy(rows0_v, out_hbm.at[pl.ds(c * NP + r0, CHUNK)])
            return carry
        lax.fori_loop(0, ROWS_PER_TILE // CHUNK, cloop, 0)

    return agg


def _make_sc_deg():
    mesh = plsc.VectorSubcoreMesh(core_axis_name="c", subcore_axis_name="s",
                                  num_cores=2, num_subcores=16)

    @functools.partial(
        pl.kernel,
        out_type=jax.ShapeDtypeStruct((2 * NP, 16), jnp.float32),
        mesh=mesh,
        compiler_params=pltpu.CompilerParams(use_tc_tiling_on_sc=False),
        scratch_types=[
            pltpu.VMEM((NCHUNK, CHUNK), jnp.int32),
            pltpu.VMEM((CHUNK, 16), jnp.float32),
            pltpu.VMEM((16, 16), jnp.float32),
            pltpu.VMEM_SHARED((NP, 16), jnp.float32),
        ],
    )
    def deg(dst_hbm, out_hbm, dst_v, ones_v, zbuf_v, acc):
        c = lax.axis_index("c")
        s = lax.axis_index("s")

        pltpu.sync_copy(dst_hbm.at[s], dst_v)
        for i in range(CHUNK):
            ones_v[i, pl.ds(0, 16)] = jnp.ones((16,), jnp.float32)
        for i in range(16):
            zbuf_v[i, pl.ds(0, 16)] = jnp.zeros((16,), jnp.float32)

        def zloop(k, carry):
            pltpu.sync_copy(zbuf_v, acc.at[pl.ds(s * ROWS_PER_TILE + k * 16, 16)])
            return carry
        lax.fori_loop(0, ROWS_PER_TILE // 16, zloop, 0)
        plsc.subcore_barrier()

        def eloop(j, carry):
            pltpu.sync_copy(ones_v, acc.at[dst_v.at[j]], add=True)
            return carry
        lax.fori_loop(c * (NCHUNK // 2), (c + 1) * (NCHUNK // 2), eloop, 0)
        plsc.subcore_barrier()

        def cloop(k, carry):
            r0 = s * ROWS_PER_TILE + k * CHUNK
            pltpu.sync_copy(acc.at[pl.ds(r0, CHUNK)], ones_v)
            pltpu.sync_copy(ones_v, out_hbm.at[pl.ds(c * NP + r0, CHUNK)])
            return carry
        lax.fori_loop(0, ROWS_PER_TILE // CHUNK, cloop, 0)

    return deg


_SC_AGG_CACHE = {}


def _sc_agg(F2):
    if F2 not in _SC_AGG_CACHE:
        _SC_AGG_CACHE[F2] = _make_sc_agg(F2)
    return _SC_AGG_CACHE[F2]


def _sc_deg():
    if 'deg' not in _SC_AGG_CACHE:
        _SC_AGG_CACHE['deg'] = _make_sc_deg()
    return _SC_AGG_CACHE['deg']


def _stack_halves(h, lo1, F2):
    return jnp.concatenate([h[:, :F2], h[:, lo1:lo1 + F2]], axis=0)


def _row_spec(F):
    return pl.BlockSpec((BLK, F), lambda i: (i, 0))


def _p_specs(F2):
    return (pl.BlockSpec((BLK, F2), lambda i: (i, 0)),
            pl.BlockSpec((BLK, F2), lambda i: (i + GRID, 0)))


def _full_spec(shape):
    nd = len(shape)
    return pl.BlockSpec(shape, lambda *_: (0,) * nd)


def _relu(v):
    return jnp.maximum(v, 0.0)


def _recomb(p0, p1, keep0):
    return jnp.concatenate([p0[:, :keep0], p1], axis=1)


def _tca_body(p0, p1, x, dinv_o, xs_o):
    deg = p0[:, 0:1] + p1[:, 0:1] + 1.0
    dinv = lax.rsqrt(deg)
    dinv_o[...] = dinv
    xs_o[...] = x[...] * dinv


def _tca(pdeg, x):
    s0, s1 = _p_specs(16)
    return pl.pallas_call(
        _tca_body,
        grid=(GRID,),
        in_specs=[s0, s1, _row_spec(128)],
        out_specs=[_row_spec(1), _row_spec(128)],
        out_shape=[jax.ShapeDtypeStruct((NP, 1), jnp.float32),
                   jax.ShapeDtypeStruct((NP, 128), jnp.float32)],
    )(pdeg, pdeg, x)


def _tcb_body(p0, p1, xs, dinv, wc, bc, hsat_o, h1s_o):
    dv = dinv[...]
    aggx = dv * (_recomb(p0[...], p1[...], 64) + xs[...])
    hcat = _relu(jnp.dot(aggx, wc[...], preferred_element_type=jnp.float32)
                 + bc[...])
    hsat_o[...] = hcat[:, :64]
    h1s_o[...] = dv * hcat[:, 64:]


def _tcb(pa, xs, dinv, wc, bc):
    s0, s1 = _p_specs(64)
    return pl.pallas_call(
        _tcb_body,
        grid=(GRID,),
        in_specs=[s0, s1, _row_spec(128), _row_spec(1),
                  _full_spec((128, 176)), _full_spec((1, 176))],
        out_specs=[_row_spec(64), _row_spec(112)],
        out_shape=[jax.ShapeDtypeStruct((NP, 64), jnp.float32),
                   jax.ShapeDtypeStruct((NP, 112), jnp.float32)],
    )(pa, pa, xs, dinv, wc, bc)


def _tcc_body(p0, p1, h1s, dinv, w1, b1, w2, t2s_o):
    dv = dinv[...]
    agg1 = dv * (_recomb(p0[...], p1[...], 48) + h1s[...])
    h2 = _relu(jnp.dot(agg1, w1[...], preferred_element_type=jnp.float32)
               + b1[...])
    t2s_o[...] = dv * jnp.dot(h2, w2[...], preferred_element_type=jnp.float32)


def _tcc(pb, h1s, dinv, w1, b1, w2):
    s0, s1 = _p_specs(64)
    return pl.pallas_call(
        _tcc_body,
        grid=(GRID,),
        in_specs=[s0, s1, _row_spec(112), _row_spec(1),
                  _full_spec((112, 420)), _full_spec((1, 420)),
                  _full_spec((420, 144))],
        out_specs=_row_spec(144),
        out_shape=jax.ShapeDtypeStruct((NP, 144), jnp.float32),
    )(pb, pb, h1s, dinv, w1, b1, w2)


def _tcd_body(p0, p1, t2s, dinv, b2, w3, t3s_o):
    dv = dinv[...]
    h3 = _relu(dv * (_recomb(p0[...], p1[...], 64) + t2s[...]) + b2[...])
    t3s_o[...] = dv * jnp.dot(h3, w3[...], preferred_element_type=jnp.float32)


def _tcd(pc, t2s, dinv, b2, w3):
    s0, s1 = _p_specs(80)
    return pl.pallas_call(
        _tcd_body,
        grid=(GRID,),
        in_specs=[s0, s1, _row_spec(144), _row_spec(1),
                  _full_spec((1, 144)), _full_spec((144, 144))],
        out_specs=_row_spec(144),
        out_shape=jax.ShapeDtypeStruct((NP, 144), jnp.float32),
    )(pc, pc, t2s, dinv, b2, w3)


def _tce_body(p0, p1, t3s, dinv, b3, hsat, bm, g_o):
    @pl.when(pl.program_id(0) == 0)
    def _():
        g_o[...] = jnp.zeros_like(g_o)

    h4 = _relu(dinv[...] * (_recomb(p0[...], p1[...], 64) + t3s[...])
               + b3[...])
    hcat = jnp.concatenate([hsat[...], h4], axis=1)
    ids = lax.broadcasted_iota(jnp.int32, (BLK, N_GRAPHS), 1).astype(jnp.float32)
    onehot = (bm[...] == ids).astype(jnp.float32)
    g_o[...] += lax.dot_general(onehot, hcat, (((0,), (0,)), ((), ())),
                                preferred_element_type=jnp.float32)


def _tce(pd, t3s, dinv, b3, hsat, bm):
    s0, s1 = _p_specs(80)
    return pl.pallas_call(
        _tce_body,
        grid=(GRID,),
        in_specs=[s0, s1, _row_spec(144), _row_spec(1),
                  _full_spec((1, 144)), _row_spec(64), _row_spec(1)],
        out_specs=_full_spec((N_GRAPHS, 208)),
        out_shape=jax.ShapeDtypeStruct((N_GRAPHS, 208), jnp.float32),
    )(pd, pd, t3s, dinv, b3, hsat, bm)


def _tcf_body(g, sat_wo, sat_bo, dw0, db0, dw1, db1, dw2, db2, dw3, db3,
              cw, cb, temp, xw, xa, pt, out_o):
    gs = _relu(g[:, :64])
    logp = (jnp.dot(gs, sat_wo[...], preferred_element_type=jnp.float32)
            + sat_bo[...]) * 0.8 + 2.5
    ln_p_amine = _LN10 * logp

    h = _relu(g[:, 64:])
    h = _relu(jnp.dot(h, dw0[...], preferred_element_type=jnp.float32) + db0[...])
    h = _relu(jnp.dot(h, dw1[...], preferred_element_type=jnp.float32) + db1[...])
    h = _relu(jnp.dot(h, dw2[...], preferred_element_type=jnp.float32) + db2[...])
    h = _relu(jnp.dot(h, dw3[...], preferred_element_type=jnp.float32) + db3[...])
    c = jnp.dot(h, cw[...], preferred_element_type=jnp.float32) + cb[...]
    c0, c1, c2 = c[:, 0:1], c[:, 1:2], c[:, 2:3]

    t = temp[...]
    x_w = xw[...]
    x_a = xa[...]

    sig = 1.0 / (1.0 + jnp.exp(-c0))
    alpha = 0.2 * (1.0 + sig / 10.0 * (0.47 / 0.2))
    rtk = 62.36367 * (t + 273.15)
    tau12 = c1 / rtk
    tau21 = c2 / rtk
    g12 = jnp.exp(-tau12 * alpha)
    g21 = jnp.exp(-tau21 * alpha)
    den21 = x_a + x_w * g21
    den12 = x_w + x_a * g12
    r21 = g21 / den21
    r12 = g12 / den12
    lga = x_w * x_w * (tau21 * r21 * r21 + tau12 * g12 / (den12 * den12))
    lgw = x_a * x_a * (tau12 * r12 * r12 + tau21 * g21 / (den21 * den21))

    ln_p_water = _LN10 * jnp.where(
        t < 100.0,
        8.07131 - 1730.63 / (t + 233.426),
        8.14019 - 1810.94 / (t + 244.485))
    ln_p_tot = jnp.log(pt[...] / 133.322)

    ln_y_amine = jnp.log(x_a) + lga + ln_p_amine - ln_p_tot
    ln_y_water = jnp.log(x_w) + lgw + ln_p_water - ln_p_tot
    out_o[...] = jnp.concatenate(
        [ln_y_amine, ln_y_water, ln_y_amine + ln_y_water], axis=1)


def _tcf(g, sat_wo, sat_bo, dws, dbs, cw, cb, temp, xw, xa, pt):
    args = [g, sat_wo, sat_bo]
    for w, b in zip(dws, dbs):
        args += [w, b]
    args += [cw, cb, temp, xw, xa, pt]
    return pl.pallas_call(
        _tcf_body,
        in_specs=[_full_spec(a.shape) for a in args],
        out_specs=_full_spec((N_GRAPHS, 3)),
        out_shape=jax.ShapeDtypeStruct((N_GRAPHS, 3), jnp.float32),
    )(*args)


def _pad2(w, rows, cols):
    return jnp.zeros((rows, cols), jnp.float32).at[:w.shape[0], :w.shape[1]].set(w)


def kernel(x, edge_index, batch_mapping, x_water, x_amine, y_water, y_amine,
           temperature, P_tot, params):
    f32 = jnp.float32

    xp = jnp.zeros((NP, 128), f32).at[:N_NODES].set(x)
    bm = jnp.full((NP, 1), 999.0, f32).at[:N_NODES, 0].set(
        batch_mapping.astype(f32))

    src = jnp.zeros((EPAD,), jnp.int32).at[:N_EDGES].set(edge_index[0])
    src = src.reshape(NSLAB, NCHUNK, CHUNK)
    src2 = jnp.stack([src, src + NP])
    dst_pad = N_NODES + (jnp.arange(EPAD - N_EDGES, dtype=jnp.int32)
                         % (NP - N_NODES))
    dst = jnp.concatenate([edge_index[1], dst_pad]).reshape(NSLAB, NCHUNK, CHUNK)

    pdeg = _sc_deg()(dst)
    dinv, xs = _tca(pdeg, xp)

    pa = _sc_agg(64)(_stack_halves(xs, 64, 64), src2, dst)
    wc = jnp.concatenate(
        [params['sat_W'], _pad2(params['gcn_W0'], 128, 112)], axis=1)
    bc = jnp.concatenate(
        [params['sat_b'], jnp.zeros((112,), f32).at[:100].set(params['gcn_b0'])]
    ).reshape(1, 176)
    hsat, h1s = _tcb(pa, xs, dinv, wc, bc)

    pb = _sc_agg(64)(_stack_halves(h1s, 48, 64), src2, dst)
    w1 = _pad2(params['gcn_W1'], 112, 420)
    b1 = params['gcn_b1'].reshape(1, 420)
    w2 = _pad2(params['gcn_W2'], 420, 144)
    t2s = _tcc(pb, h1s, dinv, w1, b1, w2)

    pc = _sc_agg(80)(_stack_halves(t2s, 64, 80), src2, dst)
    b2 = jnp.zeros((1, 144), f32).at[0, :140].set(params['gcn_b2'])
    w3 = _pad2(params['gcn_W3'], 144, 144)
    t3s = _tcd(pc, t2s, dinv, b2, w3)

    pd = _sc_agg(80)(_stack_halves(t3s, 64, 80), src2, dst)
    b3 = jnp.zeros((1, 144), f32).at[0, :140].set(params['gcn_b3'])
    g = _tce(pd, t3s, dinv, b3, hsat, bm)

    dws = [_pad2(params['dense_W0'], 144, 260), params['dense_W1'],
           params['dense_W2'], params['dense_W3']]
    dbs = [params['dense_b0'].reshape(1, 260), params['dense_b1'].reshape(1, 60),
           params['dense_b2'].reshape(1, 180), params['dense_b3'].reshape(1, 100)]
    col = lambda v: v.reshape(N_GRAPHS, 1)
    return _tcf(g, params['sat_Wo'], params['sat_bo'].reshape(1, 1),
                dws, dbs, params['coef_W'], params['coef_b'].reshape(1, 3),
                col(temperature), col(x_water), col(x_amine), col(P_tot))
